# baseline (device time: 26403 ns/iter reference)
import jax
import jax.numpy as jnp
from jax import lax
from jax.experimental import pallas as pl
from jax.experimental.pallas import tpu as pltpu

M = 1024
D = 1024
H = M // 2
C = 8
CR = H // C


def kernel(partial, resid, gamma):
    def body(
        p_hbm,
        r_hbm,
        g_ref,
        o_ref,
        p_my_buf,
        r_my_buf,
        praw_send_buf,
        praw_recv_buf,
        oq_buf,
        ox_recv_buf,
        in_sems,
        r_in_sem,
        praw_send_sems,
        praw_recv_sems,
        ox_send_sems,
        ox_recv_sems,
    ):
        my_x = lax.axis_index("x")
        my_y = lax.axis_index("y")
        y_nbr = (my_x, 1 - my_y)
        x_nbr = (1 - my_x, my_y)
        h0 = my_x * H
        h1 = (1 - my_x) * H

        barrier = pltpu.get_barrier_semaphore()
        for nbr in (y_nbr, x_nbr):
            pl.semaphore_signal(
                barrier, inc=1, device_id=nbr, device_id_type=pl.DeviceIdType.MESH
            )

        in_cp = []
        for c in range(C):
            cp = pltpu.make_async_copy(
                p_hbm.at[0, pl.ds(h0 + c * CR, CR)],
                p_my_buf.at[c],
                in_sems.at[c],
            )
            cp.start()
            in_cp.append(cp)
        r_cp = pltpu.make_async_copy(
            r_hbm.at[pl.ds(h0, H)], r_my_buf, r_in_sem
        )
        r_cp.start()

        with jax.named_scope("barrier_wait"):
            pl.semaphore_wait(barrier, 2)

        praw = []
        for c in range(C):
            in_cp[c].wait()
            praw_send_buf[c] = p_my_buf[c].astype(jnp.bfloat16)
            r_ = pltpu.make_async_remote_copy(
                src_ref=praw_send_buf.at[c],
                dst_ref=praw_recv_buf.at[c],
                send_sem=praw_send_sems.at[c],
                recv_sem=praw_recv_sems.at[c],
                device_id=y_nbr,
                device_id_type=pl.DeviceIdType.MESH,
            )
            r_.start()
            praw.append(r_)

        r_cp.wait()
        g = g_ref[...]

        ox = []
        for c in range(C):
            with jax.named_scope(f"praw_wait#c={c}"):
                praw[c].wait_recv()
            y = (
                p_my_buf[c]
                + praw_recv_buf[c].astype(jnp.float32)
                + r_my_buf[pl.ds(c * CR, CR), :]
            )
            ms = jnp.mean(y * y, axis=-1, keepdims=True) + 1e-6
            q_out = y * lax.rsqrt(ms) * g
            o_ref[pl.ds(h0 + c * CR, CR), :] = q_out
            oq_buf[c] = q_out.astype(jnp.bfloat16)
            x_ = pltpu.make_async_remote_copy(
                src_ref=oq_buf.at[c],
                dst_ref=ox_recv_buf.at[c],
                send_sem=ox_send_sems.at[c],
                recv_sem=ox_recv_sems.at[c],
                device_id=x_nbr,
                device_id_type=pl.DeviceIdType.MESH,
            )
            x_.start()
            ox.append(x_)

        for c in range(C):
            with jax.named_scope(f"ox_wait#c={c}"):
                ox[c].wait_recv()
            o_ref[pl.ds(h1 + c * CR, CR), :] = ox_recv_buf[c].astype(jnp.float32)

        with jax.named_scope("drain"):
            for c in range(C):
                praw[c].wait_send()
                ox[c].wait_send()

    return pl.pallas_call(
        body,
        out_shape=jax.ShapeDtypeStruct((M, D), jnp.float32),
        in_specs=[
            pl.BlockSpec(memory_space=pl.ANY),
            pl.BlockSpec(memory_space=pl.ANY),
            pl.BlockSpec(memory_space=pltpu.VMEM),
        ],
        out_specs=pl.BlockSpec(memory_space=pltpu.VMEM),
        scratch_shapes=[
            pltpu.VMEM((C, CR, D), jnp.float32),
            pltpu.VMEM((H, D), jnp.float32),
            pltpu.VMEM((C, CR, D), jnp.bfloat16),
            pltpu.VMEM((C, CR, D), jnp.bfloat16),
            pltpu.VMEM((C, CR, D), jnp.bfloat16),
            pltpu.VMEM((C, CR, D), jnp.bfloat16),
            pltpu.SemaphoreType.DMA((C,)),
            pltpu.SemaphoreType.DMA,
            pltpu.SemaphoreType.DMA((C,)),
            pltpu.SemaphoreType.DMA((C,)),
            pltpu.SemaphoreType.DMA((C,)),
            pltpu.SemaphoreType.DMA((C,)),
        ],
        compiler_params=pltpu.CompilerParams(collective_id=0),
    )(partial, resid, gamma)


# device time: 22604 ns/iter; 1.1681x vs baseline; 1.1681x over previous
import jax
import jax.numpy as jnp
from jax import lax
from jax.experimental import pallas as pl
from jax.experimental.pallas import tpu as pltpu

M = 1024
D = 1024
H = M // 2
C = 8
CR = H // C


def kernel(partial, resid, gamma):
    my_x = lax.axis_index("x")
    h0_out = my_x * H
    p_half = lax.dynamic_slice_in_dim(
        partial.reshape(M, D), h0_out, H
    ).astype(jnp.bfloat16)
    r_half = lax.dynamic_slice_in_dim(resid, h0_out, H).astype(jnp.bfloat16)

    def body(
        ps_ref,
        r_ref,
        g_ref,
        o_ref,
        praw_recv_buf,
        praw_send_sems,
        praw_recv_sems,
        ox_send_sems,
        ox_recv_sems,
    ):
        mx = lax.axis_index("x")
        my = lax.axis_index("y")
        y_nbr = (mx, 1 - my)
        x_nbr = (1 - mx, my)
        h0 = mx * H

        barrier = pltpu.get_barrier_semaphore()
        for nbr in (y_nbr, x_nbr):
            pl.semaphore_signal(
                barrier, inc=1, device_id=nbr, device_id_type=pl.DeviceIdType.MESH
            )
        pl.semaphore_wait(barrier, 2)

        praw = []
        for c in range(C):
            r_ = pltpu.make_async_remote_copy(
                src_ref=ps_ref.at[pl.ds(c * CR, CR)],
                dst_ref=praw_recv_buf.at[c],
                send_sem=praw_send_sems.at[c],
                recv_sem=praw_recv_sems.at[c],
                device_id=y_nbr,
                device_id_type=pl.DeviceIdType.MESH,
            )
            r_.start()
            praw.append(r_)

        g = g_ref[...]

        ox = []
        for c in range(C):
            praw[c].wait_recv()
            y = (
                ps_ref[pl.ds(c * CR, CR), :].astype(jnp.float32)
                + praw_recv_buf[c].astype(jnp.float32)
                + r_ref[pl.ds(c * CR, CR), :].astype(jnp.float32)
            )
            ms = jnp.mean(y * y, axis=-1, keepdims=True) + 1e-6
            q_out = (y * lax.rsqrt(ms) * g).astype(jnp.bfloat16)
            o_ref[pl.ds(h0 + c * CR, CR), :] = q_out
            x_ = pltpu.make_async_remote_copy(
                src_ref=o_ref.at[pl.ds(h0 + c * CR, CR)],
                dst_ref=o_ref.at[pl.ds(h0 + c * CR, CR)],
                send_sem=ox_send_sems.at[c],
                recv_sem=ox_recv_sems.at[c],
                device_id=x_nbr,
                device_id_type=pl.DeviceIdType.MESH,
            )
            x_.start()
            ox.append(x_)

        for c in range(C):
            ox[c].wait_recv()
        for c in range(C):
            praw[c].wait_send()
            ox[c].wait_send()

    return pl.pallas_call(
        body,
        out_shape=jax.ShapeDtypeStruct((M, D), jnp.bfloat16),
        in_specs=[pl.BlockSpec(memory_space=pltpu.VMEM)] * 3,
        out_specs=pl.BlockSpec(memory_space=pltpu.VMEM),
        scratch_shapes=[
            pltpu.VMEM((C, CR, D), jnp.bfloat16),
            pltpu.SemaphoreType.DMA((C,)),
            pltpu.SemaphoreType.DMA((C,)),
            pltpu.SemaphoreType.DMA((C,)),
            pltpu.SemaphoreType.DMA((C,)),
        ],
        compiler_params=pltpu.CompilerParams(collective_id=0),
    )(p_half, r_half, gamma)


# device time: 22597 ns/iter; 1.1684x vs baseline; 1.0003x over previous
import jax
import jax.numpy as jnp
from jax import lax
from jax.experimental import pallas as pl
from jax.experimental.pallas import tpu as pltpu

M = 1024
D = 1024
H = M // 2
C = 16
CR = H // C


def kernel(partial, resid, gamma):
    my_x = lax.axis_index("x")
    h0_out = my_x * H
    p_half = lax.dynamic_slice_in_dim(
        partial.reshape(M, D), h0_out, H
    ).astype(jnp.bfloat16)
    r_half = lax.dynamic_slice_in_dim(resid, h0_out, H).astype(jnp.bfloat16)

    def body(
        ps_ref,
        r_ref,
        g_ref,
        o_hbm,
        praw_recv_buf,
        oq_buf,
        ox_recv_buf,
        praw_send_sems,
        praw_recv_sems,
        ox_send_sems,
        ox_recv_sems,
        my_cp_sems,
        ox_cp_sems,
    ):
        mx = lax.axis_index("x")
        my = lax.axis_index("y")
        y_nbr = (mx, 1 - my)
        x_nbr = (1 - mx, my)
        h0 = mx * H
        h1 = (1 - mx) * H

        barrier = pltpu.get_barrier_semaphore()
        for nbr in (y_nbr, x_nbr):
            pl.semaphore_signal(
                barrier, inc=1, device_id=nbr, device_id_type=pl.DeviceIdType.MESH
            )
        pl.semaphore_wait(barrier, 2)

        praw = []
        for c in range(C):
            r_ = pltpu.make_async_remote_copy(
                src_ref=ps_ref.at[pl.ds(c * CR, CR)],
                dst_ref=praw_recv_buf.at[c],
                send_sem=praw_send_sems.at[c],
                recv_sem=praw_recv_sems.at[c],
                device_id=y_nbr,
                device_id_type=pl.DeviceIdType.MESH,
            )
            r_.start()
            praw.append(r_)

        g = g_ref[...]

        ox = []
        my_cp = []
        for c in range(C):
            praw[c].wait_recv()
            y = (
                ps_ref[pl.ds(c * CR, CR), :].astype(jnp.float32)
                + praw_recv_buf[c].astype(jnp.float32)
                + r_ref[pl.ds(c * CR, CR), :].astype(jnp.float32)
            )
            ms = jnp.mean(y * y, axis=-1, keepdims=True) + 1e-6
            oq_buf[c] = (y * lax.rsqrt(ms) * g).astype(jnp.bfloat16)
            x_ = pltpu.make_async_remote_copy(
                src_ref=oq_buf.at[c],
                dst_ref=ox_recv_buf.at[c],
                send_sem=ox_send_sems.at[c],
                recv_sem=ox_recv_sems.at[c],
                device_id=x_nbr,
                device_id_type=pl.DeviceIdType.MESH,
            )
            x_.start()
            ox.append(x_)
            cp = pltpu.make_async_copy(
                oq_buf.at[c], o_hbm.at[pl.ds(h0 + c * CR, CR)], my_cp_sems.at[c]
            )
            cp.start()
            my_cp.append(cp)

        ox_cp = []
        for c in range(C):
            ox[c].wait_recv()
            cp = pltpu.make_async_copy(
                ox_recv_buf.at[c], o_hbm.at[pl.ds(h1 + c * CR, CR)], ox_cp_sems.at[c]
            )
            cp.start()
            ox_cp.append(cp)

        for c in range(C):
            my_cp[c].wait()
            ox_cp[c].wait()
            praw[c].wait_send()
            ox[c].wait_send()

    return pl.pallas_call(
        body,
        out_shape=jax.ShapeDtypeStruct((M, D), jnp.bfloat16),
        in_specs=[pl.BlockSpec(memory_space=pltpu.VMEM)] * 3,
        out_specs=pl.BlockSpec(memory_space=pl.ANY),
        scratch_shapes=[
            pltpu.VMEM((C, CR, D), jnp.bfloat16),
            pltpu.VMEM((C, CR, D), jnp.bfloat16),
            pltpu.VMEM((C, CR, D), jnp.bfloat16),
            pltpu.SemaphoreType.DMA((C,)),
            pltpu.SemaphoreType.DMA((C,)),
            pltpu.SemaphoreType.DMA((C,)),
            pltpu.SemaphoreType.DMA((C,)),
            pltpu.SemaphoreType.DMA((C,)),
            pltpu.SemaphoreType.DMA((C,)),
        ],
        compiler_params=pltpu.CompilerParams(collective_id=0),
    )(p_half, r_half, gamma)


# device time: 22590 ns/iter; 1.1688x vs baseline; 1.0003x over previous
import jax
import jax.numpy as jnp
from jax import lax
from jax.experimental import pallas as pl
from jax.experimental.pallas import tpu as pltpu

M = 1024
D = 1024
H = M // 2
C = 8
CR = H // C


def kernel(partial, resid, gamma):
    my_x = lax.axis_index("x")
    h0_out = my_x * H
    p_half = lax.dynamic_slice_in_dim(partial[0], h0_out, H).astype(jnp.bfloat16)
    r_half = lax.dynamic_slice_in_dim(resid, h0_out, H).astype(jnp.bfloat16)

    def body(
        ps_ref,
        r_ref,
        g_ref,
        o_ref,
        praw_recv_buf,
        praw_send_sems,
        praw_recv_sems,
        ox_send_sems,
        ox_recv_sems,
    ):
        mx = lax.axis_index("x")
        my = lax.axis_index("y")
        y_nbr = (mx, 1 - my)
        x_nbr = (1 - mx, my)
        h0 = mx * H

        barrier = pltpu.get_barrier_semaphore()
        for nbr in (y_nbr, x_nbr):
            pl.semaphore_signal(
                barrier, inc=1, device_id=nbr, device_id_type=pl.DeviceIdType.MESH
            )
        pl.semaphore_wait(barrier, 2)

        praw = []
        for c in range(C):
            r_ = pltpu.make_async_remote_copy(
                src_ref=ps_ref.at[pl.ds(c * CR, CR)],
                dst_ref=praw_recv_buf.at[c],
                send_sem=praw_send_sems.at[c],
                recv_sem=praw_recv_sems.at[c],
                device_id=y_nbr,
                device_id_type=pl.DeviceIdType.MESH,
            )
            r_.start()
            praw.append(r_)

        g = g_ref[...]

        ox = []
        for c in range(C):
            praw[c].wait_recv()
            y = (
                ps_ref[pl.ds(c * CR, CR), :].astype(jnp.float32)
                + praw_recv_buf[c].astype(jnp.float32)
                + r_ref[pl.ds(c * CR, CR), :].astype(jnp.float32)
            )
            ms = jnp.mean(y * y, axis=-1, keepdims=True) + 1e-6
            q_out = (y * lax.rsqrt(ms) * g).astype(jnp.bfloat16)
            o_ref[pl.ds(h0 + c * CR, CR), :] = q_out
            x_ = pltpu.make_async_remote_copy(
                src_ref=o_ref.at[pl.ds(h0 + c * CR, CR)],
                dst_ref=o_ref.at[pl.ds(h0 + c * CR, CR)],
                send_sem=ox_send_sems.at[c],
                recv_sem=ox_recv_sems.at[c],
                device_id=x_nbr,
                device_id_type=pl.DeviceIdType.MESH,
            )
            x_.start()
            ox.append(x_)

        for c in range(C):
            ox[c].wait_recv()
        for c in range(C):
            praw[c].wait_send()
            ox[c].wait_send()

    return pl.pallas_call(
        body,
        out_shape=jax.ShapeDtypeStruct((M, D), jnp.bfloat16),
        in_specs=[pl.BlockSpec(memory_space=pltpu.VMEM)] * 3,
        out_specs=pl.BlockSpec(memory_space=pltpu.VMEM),
        scratch_shapes=[
            pltpu.VMEM((C, CR, D), jnp.bfloat16),
            pltpu.SemaphoreType.DMA((C,)),
            pltpu.SemaphoreType.DMA((C,)),
            pltpu.SemaphoreType.DMA((C,)),
            pltpu.SemaphoreType.DMA((C,)),
        ],
        compiler_params=pltpu.CompilerParams(collective_id=0),
    )(p_half, r_half, gamma)


# device time: 21561 ns/iter; 1.2246x vs baseline; 1.0477x over previous
import jax
import jax.numpy as jnp
from jax import lax
from jax.experimental import pallas as pl
from jax.experimental.pallas import tpu as pltpu

M = 1024
D = 1024
CR = 64
NC = 9
NX = 7
REG = NC * CR
EXC = NX * CR


def kernel(partial, resid, gamma):
    my_x = lax.axis_index("x")
    reg0 = my_x * EXC
    p_reg = lax.dynamic_slice_in_dim(partial[0], reg0, REG).astype(jnp.bfloat16)
    r_reg = lax.dynamic_slice_in_dim(resid, reg0, REG).astype(jnp.bfloat16)

    def body(
        ps_ref,
        r_ref,
        g_ref,
        o_ref,
        praw_recv_buf,
        praw_send_sems,
        praw_recv_sems,
        ox_send_sems,
        ox_recv_sems,
    ):
        mx = lax.axis_index("x")
        my = lax.axis_index("y")
        y_nbr = (mx, 1 - my)
        x_nbr = (1 - mx, my)

        def local_row(i):
            return CR * i + CR * (NC - 1 - 2 * i) * mx

        barrier = pltpu.get_barrier_semaphore()
        for nbr in (y_nbr, x_nbr):
            pl.semaphore_signal(
                barrier, inc=1, device_id=nbr, device_id_type=pl.DeviceIdType.MESH
            )
        pl.semaphore_wait(barrier, 2)

        praw = []
        for i in range(NC):
            li = local_row(i)
            r_ = pltpu.make_async_remote_copy(
                src_ref=ps_ref.at[pl.ds(li, CR)],
                dst_ref=praw_recv_buf.at[i],
                send_sem=praw_send_sems.at[i],
                recv_sem=praw_recv_sems.at[i],
                device_id=y_nbr,
                device_id_type=pl.DeviceIdType.MESH,
            )
            r_.start()
            praw.append(r_)

        g = g_ref[...]

        ox = []
        for i in range(NC):
            li = local_row(i)
            gi = li + EXC * mx
            praw[i].wait_recv()
            y = (
                ps_ref[pl.ds(li, CR), :].astype(jnp.float32)
                + praw_recv_buf[i].astype(jnp.float32)
                + r_ref[pl.ds(li, CR), :].astype(jnp.float32)
            )
            ms = jnp.mean(y * y, axis=-1, keepdims=True) + 1e-6
            o_ref[pl.ds(gi, CR), :] = (y * lax.rsqrt(ms) * g).astype(jnp.bfloat16)
            if i < NX:
                x_ = pltpu.make_async_remote_copy(
                    src_ref=o_ref.at[pl.ds(gi, CR)],
                    dst_ref=o_ref.at[pl.ds(gi, CR)],
                    send_sem=ox_send_sems.at[i],
                    recv_sem=ox_recv_sems.at[i],
                    device_id=x_nbr,
                    device_id_type=pl.DeviceIdType.MESH,
                )
                x_.start()
                ox.append(x_)

        for x_ in ox:
            x_.wait_recv()
        for r_ in praw:
            r_.wait_send()
        for x_ in ox:
            x_.wait_send()

    return pl.pallas_call(
        body,
        out_shape=jax.ShapeDtypeStruct((M, D), jnp.bfloat16),
        in_specs=[pl.BlockSpec(memory_space=pltpu.VMEM)] * 3,
        out_specs=pl.BlockSpec(memory_space=pltpu.VMEM),
        scratch_shapes=[
            pltpu.VMEM((NC, CR, D), jnp.bfloat16),
            pltpu.SemaphoreType.DMA((NC,)),
            pltpu.SemaphoreType.DMA((NC,)),
            pltpu.SemaphoreType.DMA((NX,)),
            pltpu.SemaphoreType.DMA((NX,)),
        ],
        compiler_params=pltpu.CompilerParams(collective_id=0),
    )(p_reg, r_reg, gamma)
